# Initial kernel scaffold; baseline (speedup 1.0000x reference)
import jax
import jax.numpy as jnp
from jax.experimental import pallas as pl


def kernel(indices, values, shape, b):
    raise NotImplementedError("write your pallas kernel here")


# speedup vs baseline: 4.3585x; 4.3585x over previous
import functools

import jax
import jax.numpy as jnp
from jax import lax
from jax.experimental import pallas as pl
from jax.experimental.pallas import tpu as pltpu
from jax.experimental.pallas import tpu_sc as plsc

NS = 16
NC = 2
LANES = 16
DH = 128
CHUNK = 80


def _spmm_body(rowr, colr, valr, b0r, b1r, zr, outr, idxr, idxc, valv, gbuf, acc,
               *, n_rows, nchunk):
    c = lax.axis_index("c")
    s = lax.axis_index("s")

    pltpu.sync_copy(rowr.at[pl.ds(s * nchunk, nchunk), :], idxr)
    pltpu.sync_copy(colr.at[pl.ds(s * nchunk, nchunk), :], idxc)
    pltpu.sync_copy(valr.at[pl.ds(s * nchunk, nchunk), :], valv)

    zrows = n_rows // NS
    pltpu.sync_copy(zr.at[pl.ds(s * zrows, zrows), :], acc.at[pl.ds(s * zrows, zrows), :])
    plsc.subcore_barrier()

    def do_half(br):
        def chunk_body(k, carry):
            pltpu.sync_copy(br.at[idxc.at[k]], gbuf.at[0])

            def group_body(g, carry2):
                vvec = valv[k, pl.ds(g * LANES, LANES)]
                for l in range(LANES):
                    vv = lax.broadcast(vvec[l], (LANES,))
                    i = g * LANES + l
                    for j in range(DH // LANES):
                        sl = pl.ds(j * LANES, LANES)
                        gbuf[0, i, sl] = gbuf[0, i, sl] * vv
                return carry2
            lax.fori_loop(0, CHUNK // LANES, group_body, 0)

            pltpu.sync_copy(gbuf.at[0], acc.at[idxr.at[k]], add=True)
            return carry
        lax.fori_loop(0, nchunk, chunk_body, 0)

    @pl.when(c == 0)
    def _():
        do_half(b0r)

    @pl.when(c == 1)
    def _():
        do_half(b1r)

    plsc.subcore_barrier()
    pltpu.sync_copy(acc.at[pl.ds(s * zrows, zrows), :], outr.at[pl.ds(s * zrows, zrows), c, :])


@jax.jit
def _spmm(row2, col2, val2, b0, b1, z):
    n_rows = b0.shape[0]
    nchunk = row2.shape[0] // NS
    mesh = plsc.VectorSubcoreMesh(core_axis_name="c", subcore_axis_name="s")
    body = functools.partial(_spmm_body, n_rows=n_rows, nchunk=nchunk)
    out = pl.kernel(
        body,
        out_type=jax.ShapeDtypeStruct((n_rows, NC, DH), jnp.float32),
        mesh=mesh,
        scratch_types=[
            pltpu.VMEM((nchunk, CHUNK), jnp.int32),
            pltpu.VMEM((nchunk, CHUNK), jnp.int32),
            pltpu.VMEM((nchunk, CHUNK), jnp.float32),
            pltpu.VMEM((2, CHUNK, DH), jnp.float32),
            pltpu.VMEM_SHARED((n_rows, DH), jnp.float32),
        ],
        compiler_params=pltpu.CompilerParams(use_tc_tiling_on_sc=False),
    )(row2, col2, val2, b0, b1, z)
    return out.reshape(n_rows, NC * DH)


def kernel(indices, values, shape, b):
    n_rows = b.shape[0]
    e = values.shape[0]
    row2 = indices[0].reshape(e // CHUNK, CHUNK)
    col2 = indices[1].reshape(e // CHUNK, CHUNK)
    val2 = values.reshape(e // CHUNK, CHUNK)
    b0 = b[:, :DH]
    b1 = b[:, DH:]
    z = jnp.zeros((n_rows, DH), jnp.float32)
    return _spmm(row2, col2, val2, b0, b1, z)


# speedup vs baseline: 7.7991x; 1.7894x over previous
import functools

import jax
import jax.numpy as jnp
from jax import lax
from jax.experimental import pallas as pl
from jax.experimental.pallas import tpu as pltpu
from jax.experimental.pallas import tpu_sc as plsc

NS = 16
NC = 2
LANES = 16
DH = 128
CHUNK = 80
NBUF = 3
BLK = 25


def _spmm_body(rowr, colr, valr, b0r, b1r, zr, outr,
               idxc, idxr, valv, gbuf, acc,
               g0, g1, g2, s0, s1, s2, tsem, zsem,
               *, n_rows, nchunk):
    c = lax.axis_index("c")
    s = lax.axis_index("s")
    gsem = (g0, g1, g2)
    ssem = (s0, s1, s2)
    nblk = nchunk // BLK

    zrows = n_rows // NS
    cp_idxc = pltpu.async_copy(colr.at[pl.ds(s * nchunk, nchunk), :], idxc, zsem)
    cp_zero = pltpu.async_copy(zr.at[pl.ds(s * zrows, zrows), :],
                               acc.at[pl.ds(s * zrows, zrows), :], zsem)
    pltpu.async_copy(rowr.at[pl.ds(s * nchunk, BLK), :], idxr.at[0], tsem)
    pltpu.async_copy(valr.at[pl.ds(s * nchunk, BLK), :], valv.at[0], tsem)
    cp_idxc.wait()
    cp_zero.wait()
    plsc.subcore_barrier()

    def do_half(br):
        def start_gather(bi, k):
            pltpu.async_copy(br.at[idxc.at[k]], gbuf.at[bi], gsem[bi])

        def wait_gather(bi, k):
            pltpu.make_async_copy(br.at[idxc.at[k]], gbuf.at[bi], gsem[bi]).wait()

        def start_scatter(bi, ring, kk):
            pltpu.async_copy(gbuf.at[bi], acc.at[idxr.at[ring, kk]], ssem[bi],
                             add=True)

        def wait_scatter(bi):
            pltpu.make_async_copy(gbuf.at[bi], acc.at[idxr.at[0, 0]],
                                  ssem[bi]).wait()

        def wait_table():
            pltpu.make_async_copy(rowr.at[pl.ds(0, BLK), :], idxr.at[0], tsem).wait()
            pltpu.make_async_copy(valr.at[pl.ds(0, BLK), :], valv.at[0], tsem).wait()

        def start_table(blk):
            ring = lax.rem(blk, 2)
            base = s * nchunk + blk * BLK
            pltpu.async_copy(rowr.at[pl.ds(base, BLK), :], idxr.at[ring], tsem)
            pltpu.async_copy(valr.at[pl.ds(base, BLK), :], valv.at[ring], tsem)

        def scale_chunk(bi, ring, kk):
            def group_body(g, carry):
                vvec = valv[ring, kk, pl.ds(g * LANES, LANES)]
                for l in range(LANES):
                    vv = lax.broadcast(vvec[l], (LANES,))
                    i = g * LANES + l
                    for j in range(DH // LANES):
                        sl = pl.ds(j * LANES, LANES)
                        gbuf[bi, i, sl] = gbuf[bi, i, sl] * vv
                return carry
            lax.fori_loop(0, CHUNK // LANES, group_body, 0)

        def slot(bi, k, t, guard_first, tail):
            blk = lax.div(k, BLK)
            kk = lax.rem(k, BLK)
            ring = lax.rem(blk, 2)

            @pl.when(kk == 0)
            def _():
                wait_table()

            wait_gather(bi, k)
            scale_chunk(bi, ring, kk)
            start_scatter(bi, ring, kk)

            bnext = (bi + 2) % NBUF
            if guard_first:
                @pl.when(t >= 1)
                def _():
                    wait_scatter(bnext)
            else:
                wait_scatter(bnext)
            if not tail:
                start_gather(bnext, k + 2)

            @pl.when((kk == 0) & (k < (nblk - 1) * BLK))
            def _():
                start_table(blk + 1)

        start_gather(0, 0)
        start_gather(1, 1)

        nmain = (nchunk - 2) // NBUF

        def iter_body(t, carry):
            for bi in range(NBUF):
                slot(bi, NBUF * t + bi, t, bi == 0, False)
            return carry
        lax.fori_loop(0, nmain, iter_body, 0)

        slot((nchunk - 2) % NBUF, nchunk - 2, nmain, False, True)
        slot((nchunk - 1) % NBUF, nchunk - 1, nmain, False, True)
        wait_scatter((nchunk - 1) % NBUF)

    @pl.when(c == 0)
    def _():
        do_half(b0r)

    @pl.when(c == 1)
    def _():
        do_half(b1r)

    plsc.subcore_barrier()
    pltpu.sync_copy(acc.at[pl.ds(s * zrows, zrows), :],
                    outr.at[pl.ds(s * zrows, zrows), c, :])


@jax.jit
def _spmm(row2, col2, val2, b0, b1, z):
    n_rows = b0.shape[0]
    nchunk = row2.shape[0] // NS
    mesh = plsc.VectorSubcoreMesh(core_axis_name="c", subcore_axis_name="s")
    body = functools.partial(_spmm_body, n_rows=n_rows, nchunk=nchunk)
    out = pl.kernel(
        body,
        out_type=jax.ShapeDtypeStruct((n_rows, NC, DH), jnp.float32),
        mesh=mesh,
        scratch_types=[
            pltpu.VMEM((nchunk, CHUNK), jnp.int32),
            pltpu.VMEM((2, BLK, CHUNK), jnp.int32),
            pltpu.VMEM((2, BLK, CHUNK), jnp.float32),
            pltpu.VMEM((NBUF, CHUNK, DH), jnp.float32),
            pltpu.VMEM_SHARED((n_rows, DH), jnp.float32),
            pltpu.SemaphoreType.DMA,
            pltpu.SemaphoreType.DMA,
            pltpu.SemaphoreType.DMA,
            pltpu.SemaphoreType.DMA,
            pltpu.SemaphoreType.DMA,
            pltpu.SemaphoreType.DMA,
            pltpu.SemaphoreType.DMA,
            pltpu.SemaphoreType.DMA,
        ],
        compiler_params=pltpu.CompilerParams(use_tc_tiling_on_sc=False),
    )(row2, col2, val2, b0, b1, z)
    return out.reshape(n_rows, NC * DH)


def kernel(indices, values, shape, b):
    n_rows = b.shape[0]
    e = values.shape[0]
    row2 = indices[0].reshape(e // CHUNK, CHUNK)
    col2 = indices[1].reshape(e // CHUNK, CHUNK)
    val2 = values.reshape(e // CHUNK, CHUNK)
    b0 = b[:, :DH]
    b1 = b[:, DH:]
    z = jnp.zeros((n_rows, DH), jnp.float32)
    return _spmm(row2, col2, val2, b0, b1, z)


# speedup vs baseline: 8.0360x; 1.0304x over previous
import functools

import jax
import jax.numpy as jnp
from jax import lax
from jax.experimental import pallas as pl
from jax.experimental.pallas import tpu as pltpu
from jax.experimental.pallas import tpu_sc as plsc

NS = 16
NC = 2
LANES = 16
DH = 128
CHUNK = 80
NBUF = 3
BLK = 25


def _spmm_body(rowr, colr, valr, br, outr,
               idxc, idxr, valv, gbuf, acc,
               g0, g1, g2, s0, s1, s2, tsem, zsem,
               *, n_rows, nchunk):
    c = lax.axis_index("c")
    s = lax.axis_index("s")
    gsem = (g0, g1, g2)
    ssem = (s0, s1, s2)
    nblk = nchunk // BLK
    nedge = nchunk * CHUNK
    zrows = n_rows // NS

    cp_idxc = pltpu.async_copy(colr.at[pl.ds(s * nchunk, nchunk), :], idxc, zsem)
    pltpu.async_copy(rowr.at[pl.ds(s * nchunk, BLK), :], idxr.at[0], tsem)
    pltpu.async_copy(valr.at[pl.ds(s * nchunk, BLK), :], valv.at[0], tsem)

    zv = jnp.zeros((LANES,), jnp.float32)

    def zrow(i, carry):
        for j in range(DH // LANES):
            gbuf[0, i, pl.ds(j * LANES, LANES)] = zv
        return carry
    lax.fori_loop(0, CHUNK, zrow, 0)
    zbase = s * zrows
    zcps = []
    nfull = zrows // CHUNK
    for r in range(nfull):
        zcps.append(pltpu.async_copy(
            gbuf.at[0], acc.at[pl.ds(zbase + r * CHUNK, CHUNK), :], s0))
    rem = zrows - nfull * CHUNK
    if rem:
        zcps.append(pltpu.async_copy(
            gbuf.at[0, pl.ds(0, rem), :],
            acc.at[pl.ds(zbase + nfull * CHUNK, rem), :], s0))

    cp_idxc.wait()
    cvec = lax.broadcast(c, (LANES,))

    def cxform(k, carry):
        for g in range(CHUNK // LANES):
            sl = pl.ds(g * LANES, LANES)
            idxc[k, sl] = idxc[k, sl] * 2 + cvec
        return carry
    lax.fori_loop(0, nchunk, cxform, 0)

    for cp in zcps:
        cp.wait()
    plsc.subcore_barrier()

    def start_gather(bi, k):
        pltpu.async_copy(br.at[idxc.at[k]], gbuf.at[bi], gsem[bi])

    def wait_gather(bi, k):
        pltpu.make_async_copy(br.at[idxc.at[k]], gbuf.at[bi], gsem[bi]).wait()

    def start_scatter(bi, ring, kk):
        pltpu.async_copy(gbuf.at[bi], acc.at[idxr.at[ring, kk]], ssem[bi],
                         add=True)

    def wait_scatter(bi):
        pltpu.make_async_copy(gbuf.at[bi], acc.at[idxr.at[0, 0]],
                              ssem[bi]).wait()

    def wait_table():
        pltpu.make_async_copy(rowr.at[pl.ds(0, BLK), :], idxr.at[0], tsem).wait()
        pltpu.make_async_copy(valr.at[pl.ds(0, BLK), :], valv.at[0], tsem).wait()

    def start_table(blk):
        ring = lax.rem(blk, 2)
        base = s * nchunk + blk * BLK
        pltpu.async_copy(rowr.at[pl.ds(base, BLK), :], idxr.at[ring], tsem)
        pltpu.async_copy(valr.at[pl.ds(base, BLK), :], valv.at[ring], tsem)

    def scale_chunk(bi, ring, kk):
        def group_body(g, carry):
            vvec = valv[ring, kk, pl.ds(g * LANES, LANES)]
            for l in range(LANES):
                vv = lax.broadcast(vvec[l], (LANES,))
                i = g * LANES + l
                for j in range(DH // LANES):
                    sl = pl.ds(j * LANES, LANES)
                    gbuf[bi, i, sl] = gbuf[bi, i, sl] * vv
            return carry
        lax.fori_loop(0, CHUNK // LANES, group_body, 0)

    def slot(bi, k, t, guard_first, tail):
        blk = lax.div(k, BLK)
        kk = lax.rem(k, BLK)
        ring = lax.rem(blk, 2)

        @pl.when(kk == 0)
        def _():
            wait_table()

        wait_gather(bi, k)
        scale_chunk(bi, ring, kk)
        start_scatter(bi, ring, kk)

        bnext = (bi + 2) % NBUF
        if guard_first:
            @pl.when(t >= 1)
            def _():
                wait_scatter(bnext)
        else:
            wait_scatter(bnext)
        if not tail:
            start_gather(bnext, k + 2)

        @pl.when((kk == 0) & (k < (nblk - 1) * BLK))
        def _():
            start_table(blk + 1)

    start_gather(0, 0)
    start_gather(1, 1)

    nmain = (nchunk - 2) // NBUF

    def iter_body(t, carry):
        for bi in range(NBUF):
            slot(bi, NBUF * t + bi, t, bi == 0, False)
        return carry
    lax.fori_loop(0, nmain, iter_body, 0)

    slot((nchunk - 2) % NBUF, nchunk - 2, nmain, False, True)
    slot((nchunk - 1) % NBUF, nchunk - 1, nmain, False, True)
    wait_scatter((nchunk - 1) % NBUF)

    plsc.subcore_barrier()
    pltpu.sync_copy(acc.at[pl.ds(zbase, zrows), :],
                    outr.at[pl.ds(zbase, zrows), c, :])


@jax.jit
def _spmm(row2, col2, val2, bview):
    n_rows = bview.shape[0] // NC
    nchunk = row2.shape[0] // NS
    mesh = plsc.VectorSubcoreMesh(core_axis_name="c", subcore_axis_name="s")
    body = functools.partial(_spmm_body, n_rows=n_rows, nchunk=nchunk)
    out = pl.kernel(
        body,
        out_type=jax.ShapeDtypeStruct((n_rows, NC, DH), jnp.float32),
        mesh=mesh,
        scratch_types=[
            pltpu.VMEM((nchunk, CHUNK), jnp.int32),
            pltpu.VMEM((2, BLK, CHUNK), jnp.int32),
            pltpu.VMEM((2, BLK, CHUNK), jnp.float32),
            pltpu.VMEM((NBUF, CHUNK, DH), jnp.float32),
            pltpu.VMEM_SHARED((n_rows, DH), jnp.float32),
            pltpu.SemaphoreType.DMA,
            pltpu.SemaphoreType.DMA,
            pltpu.SemaphoreType.DMA,
            pltpu.SemaphoreType.DMA,
            pltpu.SemaphoreType.DMA,
            pltpu.SemaphoreType.DMA,
            pltpu.SemaphoreType.DMA,
            pltpu.SemaphoreType.DMA,
        ],
        compiler_params=pltpu.CompilerParams(use_tc_tiling_on_sc=False),
    )(row2, col2, val2, bview)
    return out.reshape(n_rows, NC * DH)


def kernel(indices, values, shape, b):
    n_rows = b.shape[0]
    e = values.shape[0]
    row2 = indices[0].reshape(e // CHUNK, CHUNK)
    col2 = indices[1].reshape(e // CHUNK, CHUNK)
    val2 = values.reshape(e // CHUNK, CHUNK)
    bview = b.reshape(n_rows * NC, DH)
    return _spmm(row2, col2, val2, bview)


# speedup vs baseline: 8.1635x; 1.0159x over previous
import functools

import jax
import jax.numpy as jnp
from jax import lax
from jax.experimental import pallas as pl
from jax.experimental.pallas import tpu as pltpu
from jax.experimental.pallas import tpu_sc as plsc

NS = 16
NC = 2
LANES = 16
DH = 128
CHUNK = 80
NBUF = 3
BLK = 25


def _spmm_body(rowr, colr, valr, br, outr,
               idxc, idxr, valv, gbuf, acc,
               g0, g1, g2, s0, s1, s2, tsem, zsem,
               *, n_rows, nchunk):
    c = lax.axis_index("c")
    s = lax.axis_index("s")
    gsem = (g0, g1, g2)
    ssem = (s0, s1, s2)
    nblk = nchunk // BLK
    nedge = nchunk * CHUNK
    zrows = n_rows // NS

    cp_idxc = pltpu.async_copy(colr.at[pl.ds(s * nchunk, nchunk), :], idxc, zsem)
    pltpu.async_copy(rowr.at[pl.ds(s * nchunk, BLK), :], idxr.at[0], tsem)
    pltpu.async_copy(valr.at[pl.ds(s * nchunk, BLK), :], valv.at[0], tsem)

    zv = jnp.zeros((LANES,), jnp.float32)

    def zrow(i, carry):
        for j in range(DH // LANES):
            gbuf[0, i, pl.ds(j * LANES, LANES)] = zv
        return carry
    lax.fori_loop(0, CHUNK, zrow, 0)
    zbase = s * zrows
    zcps = []
    nfull = zrows // CHUNK
    for r in range(nfull):
        zcps.append(pltpu.async_copy(
            gbuf.at[0], acc.at[pl.ds(zbase + r * CHUNK, CHUNK), :], s0))
    rem = zrows - nfull * CHUNK
    if rem:
        zcps.append(pltpu.async_copy(
            gbuf.at[0, pl.ds(0, rem), :],
            acc.at[pl.ds(zbase + nfull * CHUNK, rem), :], s0))

    cp_idxc.wait()
    cvec = lax.broadcast(c, (LANES,))

    def cxform(k, carry):
        for g in range(CHUNK // LANES):
            sl = pl.ds(g * LANES, LANES)
            idxc[k, sl] = idxc[k, sl] * 2 + cvec
        return carry
    lax.fori_loop(0, nchunk, cxform, 0)

    for cp in zcps:
        cp.wait()
    plsc.subcore_barrier()

    dcol = c * DH

    def start_gather(bi, k):
        pltpu.async_copy(br.at[idxc.at[k]], gbuf.at[bi], gsem[bi])

    def wait_gather(bi, k):
        pltpu.make_async_copy(br.at[idxc.at[k]], gbuf.at[bi], gsem[bi]).wait()

    def start_scatter(bi, ring, kk):
        pltpu.async_copy(gbuf.at[bi], acc.at[idxr.at[ring, kk]], ssem[bi],
                         add=True)

    def wait_scatter(bi):
        pltpu.make_async_copy(gbuf.at[bi], acc.at[idxr.at[0, 0]],
                              ssem[bi]).wait()

    def wait_table():
        pltpu.make_async_copy(rowr.at[pl.ds(0, BLK), :], idxr.at[0], tsem).wait()
        pltpu.make_async_copy(valr.at[pl.ds(0, BLK), :], valv.at[0], tsem).wait()

    def start_table(blk):
        ring = lax.rem(blk, 2)
        base = s * nchunk + blk * BLK
        pltpu.async_copy(rowr.at[pl.ds(base, BLK), :], idxr.at[ring], tsem)
        pltpu.async_copy(valr.at[pl.ds(base, BLK), :], valv.at[ring], tsem)

    def scale_chunk(bi, ring, kk):
        def group_body(g, carry):
            vvec = valv[ring, kk, pl.ds(g * LANES, LANES)]
            for l in range(LANES):
                vv = lax.broadcast(vvec[l], (LANES,))
                i = g * LANES + l
                for j in range(DH // LANES):
                    sl = pl.ds(j * LANES, LANES)
                    gbuf[bi, i, sl] = gbuf[bi, i, sl] * vv
            return carry
        lax.fori_loop(0, CHUNK // LANES, group_body, 0)

    def slot(bi, k, t, guard_first, tail):
        blk = lax.div(k, BLK)
        kk = lax.rem(k, BLK)
        ring = lax.rem(blk, 2)

        @pl.when(kk == 0)
        def _():
            wait_table()

        wait_gather(bi, k)
        scale_chunk(bi, ring, kk)
        start_scatter(bi, ring, kk)

        bnext = (bi + 2) % NBUF
        if guard_first:
            @pl.when(t >= 1)
            def _():
                wait_scatter(bnext)
        else:
            wait_scatter(bnext)
        if not tail:
            start_gather(bnext, k + 2)

        @pl.when((kk == 0) & (k < (nblk - 1) * BLK))
        def _():
            start_table(blk + 1)

    start_gather(0, 0)
    start_gather(1, 1)

    nmain = (nchunk - 2) // NBUF

    def iter_body(t, carry):
        for bi in range(NBUF):
            slot(bi, NBUF * t + bi, t, bi == 0, False)
        return carry
    lax.fori_loop(0, nmain, iter_body, 0)

    slot((nchunk - 2) % NBUF, nchunk - 2, nmain, False, True)
    slot((nchunk - 1) % NBUF, nchunk - 1, nmain, False, True)
    wait_scatter((nchunk - 1) % NBUF)

    plsc.subcore_barrier()
    pltpu.sync_copy(acc.at[pl.ds(zbase, zrows), :],
                    outr.at[pl.ds(zbase, zrows), pl.ds(dcol, DH)])


@jax.jit
def _spmm(row2, col2, val2, bview):
    n_rows = bview.shape[0] // NC
    nchunk = row2.shape[0] // NS
    mesh = plsc.VectorSubcoreMesh(core_axis_name="c", subcore_axis_name="s")
    body = functools.partial(_spmm_body, n_rows=n_rows, nchunk=nchunk)
    out = pl.kernel(
        body,
        out_type=jax.ShapeDtypeStruct((n_rows, NC * DH), jnp.float32),
        mesh=mesh,
        scratch_types=[
            pltpu.VMEM((nchunk, CHUNK), jnp.int32),
            pltpu.VMEM((2, BLK, CHUNK), jnp.int32),
            pltpu.VMEM((2, BLK, CHUNK), jnp.float32),
            pltpu.VMEM((NBUF, CHUNK, DH), jnp.float32),
            pltpu.VMEM_SHARED((n_rows, DH), jnp.float32),
            pltpu.SemaphoreType.DMA,
            pltpu.SemaphoreType.DMA,
            pltpu.SemaphoreType.DMA,
            pltpu.SemaphoreType.DMA,
            pltpu.SemaphoreType.DMA,
            pltpu.SemaphoreType.DMA,
            pltpu.SemaphoreType.DMA,
            pltpu.SemaphoreType.DMA,
        ],
        compiler_params=pltpu.CompilerParams(use_tc_tiling_on_sc=False),
    )(row2, col2, val2, bview)
    return out


def kernel(indices, values, shape, b):
    n_rows = b.shape[0]
    e = values.shape[0]
    row2 = indices[0].reshape(e // CHUNK, CHUNK)
    col2 = indices[1].reshape(e // CHUNK, CHUNK)
    val2 = values.reshape(e // CHUNK, CHUNK)
    bview = b.reshape(n_rows * NC, DH)
    return _spmm(row2, col2, val2, bview)
